# baseline (device time: 28356 ns/iter reference)
import jax
import jax.numpy as jnp
from jax import lax
from jax.experimental import pallas as pl
from jax.experimental.pallas import tpu as pltpu

N_DEV = 4
B_ROWS = 8


def kernel(x):
    m, n = x.shape

    def body(x_ref, out_ref, total_ref, gath_ref, send_sems, recv_sems):
        my_pos = lax.axis_index("i")
        others = [(my_pos + k) % N_DEV for k in range(1, N_DEV)]

        barrier_sem = pltpu.get_barrier_semaphore()
        for d in others:
            pl.semaphore_signal(
                barrier_sem, inc=1,
                device_id=(d,), device_id_type=pl.DeviceIdType.MESH,
            )
        pl.semaphore_wait(barrier_sem, N_DEV - 1)

        t = x_ref[:, :]
        rows = m
        while rows > 1:
            half = rows // 2
            t = t[:half, :] * t[half:rows, :]
            rows = half
        total_ref[:, :] = t

        sends = []
        for k, d in enumerate(others):
            rdma = pltpu.make_async_remote_copy(
                src_ref=total_ref,
                dst_ref=gath_ref.at[my_pos],
                send_sem=send_sems.at[k],
                recv_sem=recv_sems.at[my_pos],
                device_id=(d,),
                device_id_type=pl.DeviceIdType.MESH,
            )
            rdma.start()
            sends.append(rdma)

        nblk = m // B_ROWS
        y = x_ref[:, :].reshape(nblk, B_ROWS, n)
        s = 1
        while s < B_ROWS:
            shifted = jnp.concatenate(
                [jnp.ones((nblk, s, n), jnp.float32), y[:, : B_ROWS - s, :]],
                axis=1,
            )
            y = y * shifted
            s *= 2

        t = y[:, B_ROWS - 1, :]
        s = 1
        while s < nblk:
            shifted = jnp.concatenate(
                [jnp.ones((s, n), jnp.float32), t[: nblk - s, :]], axis=0
            )
            t = t * shifted
            s *= 2
        blk_prefix = jnp.concatenate(
            [jnp.ones((1, n), jnp.float32), t[: nblk - 1, :]], axis=0
        )

        for d in others:
            recv = pltpu.make_async_remote_copy(
                src_ref=total_ref,
                dst_ref=gath_ref.at[d],
                send_sem=send_sems.at[0],
                recv_sem=recv_sems.at[d],
                device_id=(d,),
                device_id_type=pl.DeviceIdType.MESH,
            )
            recv.wait_recv()

        prefix = jnp.ones((1, n), dtype=jnp.float32)
        for d in range(N_DEV):
            chunk = gath_ref[d, :, :]
            prefix = prefix * jnp.where(d < my_pos, chunk, 1.0)

        out_ref[:, :] = (
            y * blk_prefix[:, None, :] * prefix[None, :, :]
        ).reshape(m, n)

        for rdma in sends:
            rdma.wait_send()

    return pl.pallas_call(
        body,
        out_shape=jax.ShapeDtypeStruct((m, n), jnp.float32),
        in_specs=[pl.BlockSpec(memory_space=pltpu.VMEM)],
        out_specs=pl.BlockSpec(memory_space=pltpu.VMEM),
        scratch_shapes=[
            pltpu.VMEM((1, n), jnp.float32),
            pltpu.VMEM((N_DEV, 1, n), jnp.float32),
            pltpu.SemaphoreType.DMA((N_DEV - 1,)),
            pltpu.SemaphoreType.DMA((N_DEV,)),
        ],
        compiler_params=pltpu.CompilerParams(collective_id=0),
    )(x)


# device time: 20151 ns/iter; 1.4072x vs baseline; 1.4072x over previous
import jax
import jax.numpy as jnp
from jax import lax
from jax.experimental import pallas as pl
from jax.experimental.pallas import tpu as pltpu

N_DEV = 4
B_ROWS = 8


def kernel(x):
    m, n = x.shape

    def body(x_ref, out_ref, total_ref, gath_ref, send_sems, recv_sems):
        my_pos = lax.axis_index("i")
        others = [(my_pos + k) % N_DEV for k in range(1, N_DEV)]

        barrier_sem = pltpu.get_barrier_semaphore()
        for d in others:
            pl.semaphore_signal(
                barrier_sem, inc=1,
                device_id=(d,), device_id_type=pl.DeviceIdType.MESH,
            )
        pl.semaphore_wait(barrier_sem, N_DEV - 1)

        t = x_ref[:, :]
        rows = m
        while rows > 1:
            half = rows // 2
            t = t[:half, :] * t[half:rows, :]
            rows = half
        total_ref[:, :] = t

        sends = []
        for k, d in enumerate(others):
            rdma = pltpu.make_async_remote_copy(
                src_ref=total_ref,
                dst_ref=gath_ref.at[my_pos],
                send_sem=send_sems.at[k],
                recv_sem=recv_sems.at[my_pos],
                device_id=(d,),
                device_id_type=pl.DeviceIdType.MESH,
            )
            rdma.start()
            sends.append(rdma)

        y = x_ref[:, :]
        s = 1
        while s < m:
            shifted = jnp.concatenate(
                [jnp.ones((s, n), jnp.float32), y[: m - s, :]], axis=0
            )
            y = y * shifted
            s *= 2

        for d in others:
            recv = pltpu.make_async_remote_copy(
                src_ref=total_ref,
                dst_ref=gath_ref.at[d],
                send_sem=send_sems.at[0],
                recv_sem=recv_sems.at[d],
                device_id=(d,),
                device_id_type=pl.DeviceIdType.MESH,
            )
            recv.wait_recv()

        prefix = jnp.ones((1, n), dtype=jnp.float32)
        for d in range(N_DEV):
            chunk = gath_ref[d, :, :]
            prefix = prefix * jnp.where(d < my_pos, chunk, 1.0)

        out_ref[:, :] = y * prefix

        for rdma in sends:
            rdma.wait_send()

    return pl.pallas_call(
        body,
        out_shape=jax.ShapeDtypeStruct((m, n), jnp.float32),
        in_specs=[pl.BlockSpec(memory_space=pltpu.VMEM)],
        out_specs=pl.BlockSpec(memory_space=pltpu.VMEM),
        scratch_shapes=[
            pltpu.VMEM((1, n), jnp.float32),
            pltpu.VMEM((N_DEV, 1, n), jnp.float32),
            pltpu.SemaphoreType.DMA((N_DEV - 1,)),
            pltpu.SemaphoreType.DMA((N_DEV,)),
        ],
        compiler_params=pltpu.CompilerParams(collective_id=0),
    )(x)


# device time: 16940 ns/iter; 1.6739x vs baseline; 1.1896x over previous
import jax
import jax.numpy as jnp
from jax import lax
from jax.experimental import pallas as pl
from jax.experimental.pallas import tpu as pltpu

N_DEV = 4
B_ROWS = 8


def kernel(x):
    m, n = x.shape

    def body(x_ref, out_ref, total_ref, gath_ref, send_sems, recv_sems):
        my_pos = lax.axis_index("i")
        others = [(my_pos + k) % N_DEV for k in range(1, N_DEV)]

        barrier_sem = pltpu.get_barrier_semaphore()
        for d in others:
            pl.semaphore_signal(
                barrier_sem, inc=1,
                device_id=(d,), device_id_type=pl.DeviceIdType.MESH,
            )
        pl.semaphore_wait(barrier_sem, N_DEV - 1)

        t = x_ref[:, :]
        rows = m
        while rows > 1:
            half = rows // 2
            t = t[:half, :] * t[half:rows, :]
            rows = half
        total_ref[:, :] = t

        sends = []
        for k, d in enumerate(others):
            rdma = pltpu.make_async_remote_copy(
                src_ref=total_ref,
                dst_ref=gath_ref.at[my_pos],
                send_sem=send_sems.at[k],
                recv_sem=recv_sems.at[my_pos],
                device_id=(d,),
                device_id_type=pl.DeviceIdType.MESH,
            )
            rdma.start()
            sends.append(rdma)

        y = x_ref[:, :]
        s = 1
        while s < m:
            shifted = jnp.concatenate(
                [jnp.ones((s, n), jnp.float32), y[: m - s, :]], axis=0
            )
            y = y * shifted
            s *= 2
        out_ref[:, :] = y

        for d in others:
            recv = pltpu.make_async_remote_copy(
                src_ref=total_ref,
                dst_ref=gath_ref.at[d],
                send_sem=send_sems.at[0],
                recv_sem=recv_sems.at[d],
                device_id=(d,),
                device_id_type=pl.DeviceIdType.MESH,
            )
            recv.wait_recv()

        prefix = jnp.ones((1, n), dtype=jnp.float32)
        for d in range(N_DEV):
            chunk = gath_ref[d, :, :]
            prefix = prefix * jnp.where(d < my_pos, chunk, 1.0)

        out_ref[:, :] = out_ref[:, :] * prefix

        for rdma in sends:
            rdma.wait_send()

    return pl.pallas_call(
        body,
        out_shape=jax.ShapeDtypeStruct((m, n), jnp.float32),
        in_specs=[pl.BlockSpec(memory_space=pltpu.VMEM)],
        out_specs=pl.BlockSpec(memory_space=pltpu.VMEM),
        scratch_shapes=[
            pltpu.VMEM((1, n), jnp.float32),
            pltpu.VMEM((N_DEV, 1, n), jnp.float32),
            pltpu.SemaphoreType.DMA((N_DEV - 1,)),
            pltpu.SemaphoreType.DMA((N_DEV,)),
        ],
        compiler_params=pltpu.CompilerParams(collective_id=0),
    )(x)


# device time: 14630 ns/iter; 1.9382x vs baseline; 1.1579x over previous
import jax
import jax.numpy as jnp
from jax import lax
from jax.experimental import pallas as pl
from jax.experimental.pallas import tpu as pltpu

N_DEV = 4
B_ROWS = 8


def kernel(x):
    m, n = x.shape

    def body(x_ref, out_ref, total_ref, gath_ref, send_sems, recv_sems):
        my_pos = lax.axis_index("i")
        others = [(my_pos + k) % N_DEV for k in range(1, N_DEV)]

        barrier_sem = pltpu.get_barrier_semaphore()
        for d in others:
            pl.semaphore_signal(
                barrier_sem, inc=1,
                device_id=(d,), device_id_type=pl.DeviceIdType.MESH,
            )
        pl.semaphore_wait(barrier_sem, N_DEV - 1)

        t = x_ref[:, :]
        rows = m
        while rows > 1:
            half = rows // 2
            t = t[:half, :] * t[half:rows, :]
            rows = half
        total_ref[:, :] = t

        sends = []
        for k, d in enumerate(others):
            rdma = pltpu.make_async_remote_copy(
                src_ref=total_ref,
                dst_ref=gath_ref.at[my_pos],
                send_sem=send_sems.at[k],
                recv_sem=recv_sems.at[my_pos],
                device_id=(d,),
                device_id_type=pl.DeviceIdType.MESH,
            )
            rdma.start()
            sends.append(rdma)

        row = lax.broadcasted_iota(jnp.int32, (m, 1), 0)
        y = x_ref[:, :]
        s = 1
        while s < m:
            shifted = pltpu.roll(y, s, 0)
            y = y * jnp.where(row < s, 1.0, shifted)
            s *= 2
        out_ref[:, :] = y

        for d in others:
            recv = pltpu.make_async_remote_copy(
                src_ref=total_ref,
                dst_ref=gath_ref.at[d],
                send_sem=send_sems.at[0],
                recv_sem=recv_sems.at[d],
                device_id=(d,),
                device_id_type=pl.DeviceIdType.MESH,
            )
            recv.wait_recv()

        prefix = jnp.ones((1, n), dtype=jnp.float32)
        for d in range(N_DEV):
            chunk = gath_ref[d, :, :]
            prefix = prefix * jnp.where(d < my_pos, chunk, 1.0)

        out_ref[:, :] = out_ref[:, :] * prefix

        for rdma in sends:
            rdma.wait_send()

    return pl.pallas_call(
        body,
        out_shape=jax.ShapeDtypeStruct((m, n), jnp.float32),
        in_specs=[pl.BlockSpec(memory_space=pltpu.VMEM)],
        out_specs=pl.BlockSpec(memory_space=pltpu.VMEM),
        scratch_shapes=[
            pltpu.VMEM((1, n), jnp.float32),
            pltpu.VMEM((N_DEV, 1, n), jnp.float32),
            pltpu.SemaphoreType.DMA((N_DEV - 1,)),
            pltpu.SemaphoreType.DMA((N_DEV,)),
        ],
        compiler_params=pltpu.CompilerParams(collective_id=0),
    )(x)
